# baseline (speedup 1.0000x reference)
import functools

import jax
import jax.numpy as jnp
from jax import lax
from jax.experimental import pallas as pl
from jax.experimental.pallas import tpu as pltpu
from jax.experimental.pallas import tpu_sc as plsc

B = 16384
D = 64
NC, NS = 2, 16
NW = NC * NS
BPW = B // NW
HPAD = 16
BB = 2048


@functools.cache
def _make_sc_gather():
    mesh = plsc.VectorSubcoreMesh(core_axis_name="c", subcore_axis_name="s",
                                  num_cores=NC, num_subcores=NS)

    @functools.partial(
        pl.kernel,
        out_type=(
            jax.ShapeDtypeStruct((B, D), jnp.float32),
            jax.ShapeDtypeStruct((B, D), jnp.float32),
        ),
        mesh=mesh,
        compiler_params=pltpu.CompilerParams(use_tc_tiling_on_sc=False),
        scratch_types=[
            pltpu.VMEM((BPW,), jnp.int32),
            pltpu.VMEM((BPW,), jnp.int32),
            pltpu.VMEM((BPW, D), jnp.float32),
            pltpu.VMEM((BPW, D), jnp.float32),
            pltpu.SemaphoreType.DMA,
            pltpu.SemaphoreType.DMA,
        ],
    )
    def _sc_gather(users_hbm, movies_hbm, U_hbm, M_hbm, eu_hbm, em_hbm,
                   uidx_v, midx_v, urows_v, mrows_v, sem_u, sem_m):
        wid = lax.axis_index("s") * NC + lax.axis_index("c")
        base = wid * BPW
        pltpu.sync_copy(users_hbm.at[pl.ds(base, BPW)], uidx_v)
        pltpu.sync_copy(movies_hbm.at[pl.ds(base, BPW)], midx_v)
        cu = pltpu.async_copy(U_hbm.at[uidx_v], urows_v, sem_u)
        cm = pltpu.async_copy(M_hbm.at[midx_v], mrows_v, sem_m)
        cu.wait()
        cm.wait()
        pltpu.sync_copy(urows_v, eu_hbm.at[pl.ds(base, BPW)])
        pltpu.sync_copy(mrows_v, em_hbm.at[pl.ds(base, BPW)])

    return _sc_gather


def _mlp_body(eu_ref, em_ref, w1u_ref, w1m_ref, b1_ref, w2_ref, out_ref):
    h = jnp.dot(eu_ref[...], w1u_ref[...], preferred_element_type=jnp.float32)
    h = h + jnp.dot(em_ref[...], w1m_ref[...],
                    preferred_element_type=jnp.float32)
    h = jnp.maximum(h + b1_ref[...], 0.0)
    y = jnp.sum(h * w2_ref[...], axis=1)
    out_ref[...] = jax.nn.sigmoid(y) * 6.0 - 0.5


def kernel(users, movies, U, M, W1, b1, W2, b2):
    users = users.astype(jnp.int32)
    movies = movies.astype(jnp.int32)

    w1u = jnp.zeros((D, HPAD), jnp.float32).at[:, :10].set(W1[:, :D].T)
    w1m = jnp.zeros((D, HPAD), jnp.float32).at[:, :10].set(W1[:, D:].T)
    b1p = jnp.zeros((1, HPAD), jnp.float32).at[0, :10].set(b1).at[0, 10].set(1.0)
    w2p = jnp.zeros((1, HPAD), jnp.float32).at[0, :10].set(W2[0]).at[0, 10].set(b2[0])

    eu, em = _make_sc_gather()(users, movies, U, M)

    out2d = pl.pallas_call(
        _mlp_body,
        grid=(B // BB,),
        in_specs=[
            pl.BlockSpec((BB, D), lambda i: (i, 0)),
            pl.BlockSpec((BB, D), lambda i: (i, 0)),
            pl.BlockSpec((D, HPAD), lambda i: (0, 0)),
            pl.BlockSpec((D, HPAD), lambda i: (0, 0)),
            pl.BlockSpec((1, HPAD), lambda i: (0, 0)),
            pl.BlockSpec((1, HPAD), lambda i: (0, 0)),
        ],
        out_specs=pl.BlockSpec((BB,), lambda i: (i,)),
        out_shape=jax.ShapeDtypeStruct((B,), jnp.float32),
    )(eu, em, w1u, w1m, b1p, w2p)
    return out2d


# speedup vs baseline: 2.1032x; 2.1032x over previous
import functools

import jax
import jax.numpy as jnp
from jax import lax
from jax.experimental import pallas as pl
from jax.experimental.pallas import tpu as pltpu
from jax.experimental.pallas import tpu_sc as plsc

B = 16384
D = 64
R = 1_000_000
NC, NS = 2, 16
NW = NC * NS
BPW = B // NW
L = 16
BLK = 8192
NBLK = (R + BLK - 1) // BLK
NZ = NBLK * (BLK // 8)


def _stage1_body(ut_ref, mt_ref, w1u_ref, w1m_ref, zu_ref, zm_ref):
    zut = jax.lax.dot_general(ut_ref[...], w1u_ref[...],
                              (((0,), (1,)), ((), ())),
                              preferred_element_type=jnp.float32)
    zmt = jax.lax.dot_general(mt_ref[...], w1m_ref[...],
                              (((0,), (1,)), ((), ())),
                              preferred_element_type=jnp.float32)
    for g in range(BLK // 1024):
        up = [zut[(g * 8 + k) * 128:(g * 8 + k + 1) * 128, :]
              for k in range(8)]
        zu_ref[g * 128:(g + 1) * 128, :] = jnp.concatenate(up, axis=1)
        mp = [zmt[(g * 8 + k) * 128:(g * 8 + k + 1) * 128, :]
              for k in range(8)]
        zm_ref[g * 128:(g + 1) * 128, :] = jnp.concatenate(mp, axis=1)


@functools.cache
def _make_stage2():
    mesh = plsc.VectorSubcoreMesh(core_axis_name="c", subcore_axis_name="s",
                                  num_cores=NC, num_subcores=NS)

    @functools.partial(
        pl.kernel,
        out_type=jax.ShapeDtypeStruct((B,), jnp.float32),
        mesh=mesh,
        compiler_params=pltpu.CompilerParams(needs_layout_passes=False),
        scratch_types=[
            pltpu.VMEM((BPW,), jnp.int32),
            pltpu.VMEM((BPW,), jnp.int32),
            pltpu.VMEM((BPW,), jnp.int32),
            pltpu.VMEM((BPW,), jnp.int32),
            pltpu.VMEM((BPW,), jnp.int32),
            pltpu.VMEM((BPW,), jnp.int32),
            pltpu.VMEM((BPW // 2, 128), jnp.float32),
            pltpu.VMEM((BPW, L), jnp.float32),
            pltpu.VMEM((BPW,), jnp.float32),
            pltpu.VMEM((L,), jnp.float32),
            pltpu.VMEM((L,), jnp.float32),
            pltpu.SemaphoreType.DMA,
        ],
    )
    def _stage2(users_hbm, movies_hbm, zu_hbm, zm_hbm, b1_hbm, w2_hbm,
                out_hbm, uidx_v, midx_v, urow_v, mrow_v, uslot_v, mslot_v,
                big_v, upiece_v, out_v, b1_v, w2_v, sem):
        wid = lax.axis_index("s") * NC + lax.axis_index("c")
        base = wid * BPW
        pltpu.sync_copy(users_hbm.at[pl.ds(base, BPW)], uidx_v)
        pltpu.sync_copy(movies_hbm.at[pl.ds(base, BPW)], midx_v)
        pltpu.sync_copy(b1_hbm, b1_v)
        pltpu.sync_copy(w2_hbm, w2_v)

        def prep(ec, carry):
            ru = uidx_v[pl.ds(ec * L, L)]
            urow_v[pl.ds(ec * L, L)] = ((ru >> 10) << 7) + (ru & 127)
            uslot_v[pl.ds(ec * L, L)] = ((ru >> 7) & 7) << 4
            rm = midx_v[pl.ds(ec * L, L)]
            mrow_v[pl.ds(ec * L, L)] = ((rm >> 10) << 7) + (rm & 127)
            mslot_v[pl.ds(ec * L, L)] = ((rm >> 7) & 7) << 4
            return carry
        lax.fori_loop(0, BPW // L, prep, 0)

        HB = BPW // 2

        for half in range(2):
            h0 = half * HB
            pltpu.async_copy(
                zu_hbm.at[urow_v.at[pl.ds(h0, HB)]], big_v, sem).wait()

            def extract_u(ec, carry):
                svec = uslot_v[pl.ds(h0 + ec * L, L)]
                for k in range(L):
                    e = ec * L + k
                    upiece_v[h0 + e, :] = big_v[e, pl.ds(svec[k], L)]
                return carry
            lax.fori_loop(0, HB // L, extract_u, 0)

        b1vec = b1_v[...]
        w2vec = w2_v[...]
        lanes = lax.iota(jnp.int32, L)

        for half in range(2):
            h0 = half * HB
            pltpu.async_copy(
                zm_hbm.at[mrow_v.at[pl.ds(h0, HB)]], big_v, sem).wait()

            def finish(ec, carry):
                svec = mslot_v[pl.ds(h0 + ec * L, L)]
                acc = jnp.zeros((L,), jnp.float32)
                for k in range(L):
                    e = ec * L + k
                    mpiece = big_v[e, pl.ds(svec[k], L)]
                    h = jnp.maximum(upiece_v[h0 + e, :] + mpiece + b1vec, 0.0)
                    y = jnp.sum(h * w2vec, axis=0)
                    acc = jnp.where(lanes == k, y, acc)
                out_v[pl.ds(h0 + ec * L, L)] = \
                    6.0 / (1.0 + jnp.exp(-acc)) - 0.5
                return carry
            lax.fori_loop(0, HB // L, finish, 0)

        pltpu.sync_copy(out_v, out_hbm.at[pl.ds(base, BPW)])

    return _stage2


def kernel(users, movies, U, M, W1, b1, W2, b2):
    users = users.astype(jnp.int32)
    movies = movies.astype(jnp.int32)

    w1u = jnp.zeros((L, D), jnp.float32).at[:10, :].set(W1[:, :D])
    w1m = jnp.zeros((L, D), jnp.float32).at[:10, :].set(W1[:, D:])
    b1p = jnp.zeros((L,), jnp.float32).at[:10].set(b1).at[10].set(1.0)
    w2p = jnp.zeros((L,), jnp.float32).at[:10].set(W2[0]).at[10].set(b2[0])

    zu, zm = pl.pallas_call(
        _stage1_body,
        grid=(NBLK,),
        in_specs=[
            pl.BlockSpec((D, BLK), lambda i: (0, i)),
            pl.BlockSpec((D, BLK), lambda i: (0, i)),
            pl.BlockSpec((L, D), lambda i: (0, 0)),
            pl.BlockSpec((L, D), lambda i: (0, 0)),
        ],
        out_specs=[
            pl.BlockSpec((BLK // 8, 128), lambda i: (i, 0)),
            pl.BlockSpec((BLK // 8, 128), lambda i: (i, 0)),
        ],
        out_shape=[
            jax.ShapeDtypeStruct((NZ, 128), jnp.float32),
            jax.ShapeDtypeStruct((NZ, 128), jnp.float32),
        ],
    )(U.T, M.T, w1u, w1m)

    return _make_stage2()(users, movies, zu, zm, b1p, w2p)


# speedup vs baseline: 2.6431x; 1.2567x over previous
import functools

import jax
import jax.numpy as jnp
from jax import lax
from jax.experimental import pallas as pl
from jax.experimental.pallas import tpu as pltpu
from jax.experimental.pallas import tpu_sc as plsc

B = 16384
D = 64
R = 1_000_000
NC, NS = 2, 16
NW = NC * NS
BPW = B // NW
L = 16
BLK = 8192
NBLK = (R + BLK - 1) // BLK
NZ = NBLK * (BLK // 8)


def _stage1_body(ut_ref, mt_ref, w1_ref, zu_ref, zm_ref):
    for g in range(BLK // 1024):
        cols = pl.ds(g * 1024, 1024)
        zz = jnp.dot(w1_ref[...],
                     jnp.concatenate([ut_ref[:, cols], mt_ref[:, cols]],
                                     axis=0),
                     preferred_element_type=jnp.float32)
        t = [jnp.transpose(zz[:, k * 128:(k + 1) * 128]) for k in range(8)]
        rows = pl.ds(g * 128, 128)
        zu_ref[rows, :] = jnp.concatenate([p[:, :16] for p in t], axis=1)
        zm_ref[rows, :] = jnp.concatenate([p[:, 16:] for p in t], axis=1)


@functools.cache
def _make_stage2():
    mesh = plsc.VectorSubcoreMesh(core_axis_name="c", subcore_axis_name="s",
                                  num_cores=NC, num_subcores=NS)

    @functools.partial(
        pl.kernel,
        out_type=jax.ShapeDtypeStruct((B,), jnp.float32),
        mesh=mesh,
        compiler_params=pltpu.CompilerParams(needs_layout_passes=False),
        scratch_types=[
            pltpu.VMEM((BPW,), jnp.int32),
            pltpu.VMEM((BPW,), jnp.int32),
            pltpu.VMEM((BPW,), jnp.int32),
            pltpu.VMEM((BPW,), jnp.int32),
            pltpu.VMEM((BPW,), jnp.int32),
            pltpu.VMEM((BPW,), jnp.int32),
            pltpu.VMEM((BPW // 2, 128), jnp.float32),
            pltpu.VMEM((BPW, L), jnp.float32),
            pltpu.VMEM((BPW,), jnp.float32),
            pltpu.VMEM((L,), jnp.float32),
            pltpu.VMEM((L,), jnp.float32),
            pltpu.SemaphoreType.DMA,
        ],
    )
    def _stage2(users_hbm, movies_hbm, zu_hbm, zm_hbm, b1_hbm, w2_hbm,
                out_hbm, uidx_v, midx_v, urow_v, mrow_v, uslot_v, mslot_v,
                big_v, upiece_v, out_v, b1_v, w2_v, sem):
        wid = lax.axis_index("s") * NC + lax.axis_index("c")
        base = wid * BPW
        pltpu.sync_copy(users_hbm.at[pl.ds(base, BPW)], uidx_v)
        pltpu.sync_copy(movies_hbm.at[pl.ds(base, BPW)], midx_v)
        pltpu.sync_copy(b1_hbm, b1_v)
        pltpu.sync_copy(w2_hbm, w2_v)

        def prep(ec, carry):
            ru = uidx_v[pl.ds(ec * L, L)]
            urow_v[pl.ds(ec * L, L)] = ((ru >> 10) << 7) + (ru & 127)
            uslot_v[pl.ds(ec * L, L)] = ((ru >> 7) & 7) << 4
            rm = midx_v[pl.ds(ec * L, L)]
            mrow_v[pl.ds(ec * L, L)] = ((rm >> 10) << 7) + (rm & 127)
            mslot_v[pl.ds(ec * L, L)] = ((rm >> 7) & 7) << 4
            return carry
        lax.fori_loop(0, BPW // L, prep, 0)

        HB = BPW // 2

        for half in range(2):
            h0 = half * HB
            pltpu.async_copy(
                zu_hbm.at[urow_v.at[pl.ds(h0, HB)]], big_v, sem).wait()

            def extract_u(ec, carry):
                svec = uslot_v[pl.ds(h0 + ec * L, L)]
                for k in range(L):
                    e = ec * L + k
                    upiece_v[h0 + e, :] = big_v[e, pl.ds(svec[k], L)]
                return carry
            lax.fori_loop(0, HB // L, extract_u, 0)

        b1vec = b1_v[...]
        w2vec = w2_v[...]
        lanes = lax.iota(jnp.int32, L)

        for half in range(2):
            h0 = half * HB
            pltpu.async_copy(
                zm_hbm.at[mrow_v.at[pl.ds(h0, HB)]], big_v, sem).wait()

            def finish(ec, carry):
                svec = mslot_v[pl.ds(h0 + ec * L, L)]
                acc = jnp.zeros((L,), jnp.float32)
                for k in range(L):
                    e = ec * L + k
                    mpiece = big_v[e, pl.ds(svec[k], L)]
                    h = jnp.maximum(upiece_v[h0 + e, :] + mpiece + b1vec, 0.0)
                    y = jnp.sum(h * w2vec, axis=0)
                    acc = jnp.where(lanes == k, y, acc)
                out_v[pl.ds(h0 + ec * L, L)] = \
                    6.0 / (1.0 + jnp.exp(-acc)) - 0.5
                return carry
            lax.fori_loop(0, HB // L, finish, 0)

        pltpu.sync_copy(out_v, out_hbm.at[pl.ds(base, BPW)])

    return _stage2


def kernel(users, movies, U, M, W1, b1, W2, b2):
    users = users.astype(jnp.int32)
    movies = movies.astype(jnp.int32)

    w1c = (jnp.zeros((2 * L, 2 * D), jnp.float32)
           .at[:10, :D].set(W1[:, :D])
           .at[L:L + 10, D:].set(W1[:, D:]))
    b1p = jnp.zeros((L,), jnp.float32).at[:10].set(b1).at[10].set(1.0)
    w2p = jnp.zeros((L,), jnp.float32).at[:10].set(W2[0]).at[10].set(b2[0])

    zu, zm = pl.pallas_call(
        _stage1_body,
        grid=(NBLK,),
        compiler_params=pltpu.CompilerParams(
            fuse_transposed_lhs_in_matmul=True),
        in_specs=[
            pl.BlockSpec((D, BLK), lambda i: (0, i)),
            pl.BlockSpec((D, BLK), lambda i: (0, i)),
            pl.BlockSpec((2 * L, 2 * D), lambda i: (0, 0)),
        ],
        out_specs=[
            pl.BlockSpec((BLK // 8, 128), lambda i: (i, 0)),
            pl.BlockSpec((BLK // 8, 128), lambda i: (i, 0)),
        ],
        out_shape=[
            jax.ShapeDtypeStruct((NZ, 128), jnp.float32),
            jax.ShapeDtypeStruct((NZ, 128), jnp.float32),
        ],
    )(U.T, M.T, w1c)

    return _make_stage2()(users, movies, zu, zm, b1p, w2p)


# speedup vs baseline: 4.6983x; 1.7776x over previous
import functools

import jax
import jax.numpy as jnp
from jax import lax
from jax.experimental import pallas as pl
from jax.experimental.pallas import tpu as pltpu
from jax.experimental.pallas import tpu_sc as plsc

B = 16384
D = 64
R = 1_000_000
NC, NS = 2, 16
NW = NC * NS
BPW = B // NW
L = 16
BLK = 8192
NBLK = (R + BLK - 1) // BLK
NZ = NBLK * (BLK // 4)


def _stage1_body(ut_ref, mt_ref, w1_ref, zc_ref):
    for g in range(BLK // 512):
        cols = pl.ds(g * 512, 512)
        zz = jnp.dot(w1_ref[...],
                     jnp.concatenate([ut_ref[:, cols], mt_ref[:, cols]],
                                     axis=0),
                     preferred_element_type=jnp.float32)
        big = jnp.concatenate([zz[:, i * 128:(i + 1) * 128] for i in range(4)],
                              axis=0)
        zc_ref[pl.ds(g * 128, 128), :] = jnp.transpose(big)


@functools.cache
def _make_stage2():
    mesh = plsc.VectorSubcoreMesh(core_axis_name="c", subcore_axis_name="s",
                                  num_cores=NC, num_subcores=NS)

    @functools.partial(
        pl.kernel,
        out_type=jax.ShapeDtypeStruct((B,), jnp.float32),
        mesh=mesh,
        compiler_params=pltpu.CompilerParams(needs_layout_passes=False),
        scratch_types=[
            pltpu.VMEM((BPW,), jnp.int32),
            pltpu.VMEM((BPW,), jnp.int32),
            pltpu.VMEM((BPW,), jnp.int32),
            pltpu.VMEM((BPW,), jnp.int32),
            pltpu.VMEM((BPW,), jnp.int32),
            pltpu.VMEM((BPW,), jnp.int32),
            pltpu.VMEM((BPW // 2, 128), jnp.float32),
            pltpu.VMEM((BPW, L), jnp.float32),
            pltpu.VMEM((BPW,), jnp.float32),
            pltpu.VMEM((L,), jnp.float32),
            pltpu.VMEM((L,), jnp.float32),
            pltpu.SemaphoreType.DMA,
        ],
    )
    def _stage2(users_hbm, movies_hbm, zc_hbm, b1_hbm, w2_hbm,
                out_hbm, uidx_v, midx_v, urow_v, mrow_v, uslot_v, mslot_v,
                big_v, upiece_v, out_v, b1_v, w2_v, sem):
        wid = lax.axis_index("s") * NC + lax.axis_index("c")
        base = wid * BPW
        pltpu.sync_copy(users_hbm.at[pl.ds(base, BPW)], uidx_v)
        pltpu.sync_copy(movies_hbm.at[pl.ds(base, BPW)], midx_v)
        pltpu.sync_copy(b1_hbm, b1_v)
        pltpu.sync_copy(w2_hbm, w2_v)

        def prep(ec, carry):
            ru = uidx_v[pl.ds(ec * L, L)]
            urow_v[pl.ds(ec * L, L)] = ((ru >> 9) << 7) + (ru & 127)
            uslot_v[pl.ds(ec * L, L)] = ((ru >> 7) & 3) << 5
            rm = midx_v[pl.ds(ec * L, L)]
            mrow_v[pl.ds(ec * L, L)] = ((rm >> 9) << 7) + (rm & 127)
            mslot_v[pl.ds(ec * L, L)] = (((rm >> 7) & 3) << 5) + L
            return carry
        lax.fori_loop(0, BPW // L, prep, 0)

        HB = BPW // 2

        for half in range(2):
            h0 = half * HB
            pltpu.async_copy(
                zc_hbm.at[urow_v.at[pl.ds(h0, HB)]], big_v, sem).wait()

            def extract_u(ec, carry):
                svec = uslot_v[pl.ds(h0 + ec * L, L)]
                for k in range(L):
                    e = ec * L + k
                    upiece_v[h0 + e, :] = big_v[e, pl.ds(svec[k], L)]
                return carry
            lax.fori_loop(0, HB // L, extract_u, 0)

        b1vec = b1_v[...]
        w2vec = w2_v[...]
        lanes = lax.iota(jnp.int32, L)

        for half in range(2):
            h0 = half * HB
            pltpu.async_copy(
                zc_hbm.at[mrow_v.at[pl.ds(h0, HB)]], big_v, sem).wait()

            def finish(ec, carry):
                svec = mslot_v[pl.ds(h0 + ec * L, L)]
                acc = jnp.zeros((L,), jnp.float32)
                for k in range(L):
                    e = ec * L + k
                    mpiece = big_v[e, pl.ds(svec[k], L)]
                    h = jnp.maximum(upiece_v[h0 + e, :] + mpiece + b1vec, 0.0)
                    y = jnp.sum(h * w2vec, axis=0)
                    acc = jnp.where(lanes == k, y, acc)
                out_v[pl.ds(h0 + ec * L, L)] = \
                    6.0 / (1.0 + jnp.exp(-acc)) - 0.5
                return carry
            lax.fori_loop(0, HB // L, finish, 0)

        pltpu.sync_copy(out_v, out_hbm.at[pl.ds(base, BPW)])

    return _stage2


def kernel(users, movies, U, M, W1, b1, W2, b2):
    users = users.astype(jnp.int32)
    movies = movies.astype(jnp.int32)

    w1c = (jnp.zeros((2 * L, 2 * D), jnp.float32)
           .at[:10, :D].set(W1[:, :D])
           .at[L:L + 10, D:].set(W1[:, D:]))
    b1p = jnp.zeros((L,), jnp.float32).at[:10].set(b1).at[10].set(1.0)
    w2p = jnp.zeros((L,), jnp.float32).at[:10].set(W2[0]).at[10].set(b2[0])

    zc = pl.pallas_call(
        _stage1_body,
        grid=(NBLK,),
        compiler_params=pltpu.CompilerParams(
            fuse_transposed_lhs_in_matmul=True),
        in_specs=[
            pl.BlockSpec((D, BLK), lambda i: (0, i)),
            pl.BlockSpec((D, BLK), lambda i: (0, i)),
            pl.BlockSpec((2 * L, 2 * D), lambda i: (0, 0)),
        ],
        out_specs=pl.BlockSpec((BLK // 4, 128), lambda i: (i, 0)),
        out_shape=jax.ShapeDtypeStruct((NZ, 128), jnp.float32),
    )(U.T, M.T, w1c)

    return _make_stage2()(users, movies, zc, b1p, w2p)


# speedup vs baseline: 4.9916x; 1.0624x over previous
import functools

import jax
import jax.numpy as jnp
from jax import lax
from jax.experimental import pallas as pl
from jax.experimental.pallas import tpu as pltpu
from jax.experimental.pallas import tpu_sc as plsc

B = 16384
D = 64
R = 1_000_000
NC, NS = 2, 16
NW = NC * NS
BPW = B // NW
L = 16
BLK = 8192
NBLK = (R + BLK - 1) // BLK
NZ = NBLK * (BLK // 8)


def _stage1_body(ut_ref, mt_ref, w1_ref, zc_ref):
    for g in range(BLK // 1024):
        cols = pl.ds(g * 1024, 1024)
        zz = jnp.dot(w1_ref[...],
                     jnp.concatenate([ut_ref[:, cols], mt_ref[:, cols]],
                                     axis=0),
                     preferred_element_type=jnp.float32)
        au = jax.lax.bitcast_convert_type(
            zz[:16, :].astype(jnp.bfloat16), jnp.uint16).astype(jnp.uint32)
        bm = jax.lax.bitcast_convert_type(
            zz[16:, :].astype(jnp.bfloat16), jnp.uint16).astype(jnp.uint32)
        packed = (au | (bm << 16)).astype(jnp.int32)
        big = jnp.concatenate(
            [packed[:, i * 128:(i + 1) * 128] for i in range(8)],
            axis=0)
        zc_ref[pl.ds(g * 128, 128), :] = jnp.transpose(big)


@functools.cache
def _make_stage2():
    mesh = plsc.VectorSubcoreMesh(core_axis_name="c", subcore_axis_name="s",
                                  num_cores=NC, num_subcores=NS)

    @functools.partial(
        pl.kernel,
        out_type=jax.ShapeDtypeStruct((B,), jnp.float32),
        mesh=mesh,
        compiler_params=pltpu.CompilerParams(needs_layout_passes=False),
        scratch_types=[
            pltpu.VMEM((BPW,), jnp.int32),
            pltpu.VMEM((BPW,), jnp.int32),
            pltpu.VMEM((BPW,), jnp.int32),
            pltpu.VMEM((BPW,), jnp.int32),
            pltpu.VMEM((BPW,), jnp.int32),
            pltpu.VMEM((BPW,), jnp.int32),
            pltpu.VMEM((BPW // 2, 128), jnp.int32),
            pltpu.VMEM((BPW, L), jnp.float32),
            pltpu.VMEM((BPW,), jnp.float32),
            pltpu.VMEM((L,), jnp.float32),
            pltpu.VMEM((L,), jnp.float32),
            pltpu.SemaphoreType.DMA,
        ],
    )
    def _stage2(users_hbm, movies_hbm, zc_hbm, b1_hbm, w2_hbm,
                out_hbm, uidx_v, midx_v, urow_v, mrow_v, uslot_v, mslot_v,
                big_v, upiece_v, out_v, b1_v, w2_v, sem):
        wid = lax.axis_index("s") * NC + lax.axis_index("c")
        base = wid * BPW
        pltpu.sync_copy(users_hbm.at[pl.ds(base, BPW)], uidx_v)
        pltpu.sync_copy(movies_hbm.at[pl.ds(base, BPW)], midx_v)
        pltpu.sync_copy(b1_hbm, b1_v)
        pltpu.sync_copy(w2_hbm, w2_v)

        def prep(ec, carry):
            ru = uidx_v[pl.ds(ec * L, L)]
            urow_v[pl.ds(ec * L, L)] = ((ru >> 10) << 7) + (ru & 127)
            uslot_v[pl.ds(ec * L, L)] = ((ru >> 7) & 7) << 4
            rm = midx_v[pl.ds(ec * L, L)]
            mrow_v[pl.ds(ec * L, L)] = ((rm >> 10) << 7) + (rm & 127)
            mslot_v[pl.ds(ec * L, L)] = ((rm >> 7) & 7) << 4
            return carry
        lax.fori_loop(0, BPW // L, prep, 0)

        HB = BPW // 2

        for half in range(2):
            h0 = half * HB
            pltpu.async_copy(
                zc_hbm.at[urow_v.at[pl.ds(h0, HB)]], big_v, sem).wait()

            def extract_u(ec, carry):
                svec = uslot_v[pl.ds(h0 + ec * L, L)]
                for k in range(L):
                    e = ec * L + k
                    chunk = plsc.bitcast(big_v[e, pl.ds(svec[k], L)],
                                         jnp.bfloat16)
                    zu_e, _ = plsc.unpack(chunk,
                                          format=plsc.PackFormat.INTERLEAVED)
                    upiece_v[h0 + e, :] = zu_e
                return carry
            lax.fori_loop(0, HB // L, extract_u, 0)

        b1vec = b1_v[...]
        w2vec = w2_v[...]
        lanes = lax.iota(jnp.int32, L)

        for half in range(2):
            h0 = half * HB
            pltpu.async_copy(
                zc_hbm.at[mrow_v.at[pl.ds(h0, HB)]], big_v, sem).wait()

            def finish(ec, carry):
                svec = mslot_v[pl.ds(h0 + ec * L, L)]
                acc = jnp.zeros((L,), jnp.float32)
                for k in range(L):
                    e = ec * L + k
                    chunk = plsc.bitcast(big_v[e, pl.ds(svec[k], L)],
                                         jnp.bfloat16)
                    _, mpiece = plsc.unpack(
                        chunk, format=plsc.PackFormat.INTERLEAVED)
                    h = jnp.maximum(upiece_v[h0 + e, :] + mpiece + b1vec, 0.0)
                    y = jnp.sum(h * w2vec, axis=0)
                    acc = jnp.where(lanes == k, y, acc)
                out_v[pl.ds(h0 + ec * L, L)] = \
                    6.0 / (1.0 + jnp.exp(-acc)) - 0.5
                return carry
            lax.fori_loop(0, HB // L, finish, 0)

        pltpu.sync_copy(out_v, out_hbm.at[pl.ds(base, BPW)])

    return _stage2


def kernel(users, movies, U, M, W1, b1, W2, b2):
    users = users.astype(jnp.int32)
    movies = movies.astype(jnp.int32)

    w1c = (jnp.zeros((2 * L, 2 * D), jnp.float32)
           .at[:10, :D].set(W1[:, :D])
           .at[L:L + 10, D:].set(W1[:, D:]))
    b1p = jnp.zeros((L,), jnp.float32).at[:10].set(b1).at[10].set(1.0)
    w2p = jnp.zeros((L,), jnp.float32).at[:10].set(W2[0]).at[10].set(b2[0])

    zc = pl.pallas_call(
        _stage1_body,
        grid=(NBLK,),
        compiler_params=pltpu.CompilerParams(
            fuse_transposed_lhs_in_matmul=True),
        in_specs=[
            pl.BlockSpec((D, BLK), lambda i: (0, i)),
            pl.BlockSpec((D, BLK), lambda i: (0, i)),
            pl.BlockSpec((2 * L, 2 * D), lambda i: (0, 0)),
        ],
        out_specs=pl.BlockSpec((BLK // 8, 128), lambda i: (i, 0)),
        out_shape=jax.ShapeDtypeStruct((NZ, 128), jnp.int32),
    )(U.T, M.T, w1c)

    return _make_stage2()(users, movies, zc, b1p, w2p)


# speedup vs baseline: 5.0151x; 1.0047x over previous
import functools

import jax
import jax.numpy as jnp
from jax import lax
from jax.experimental import pallas as pl
from jax.experimental.pallas import tpu as pltpu
from jax.experimental.pallas import tpu_sc as plsc

B = 16384
D = 64
R = 1_000_000
NC, NS = 2, 16
NW = NC * NS
BPW = B // NW
L = 16
BLK = 8192
NBLK = (R + BLK - 1) // BLK
NZ = NBLK * (BLK // 8)


def _stage1_body(ut_ref, mt_ref, w1_ref, zc_ref):
    for g in range(BLK // 1024):
        cols = pl.ds(g * 1024, 1024)
        zz = jnp.dot(w1_ref[...],
                     jnp.concatenate([ut_ref[:, cols], mt_ref[:, cols]],
                                     axis=0),
                     preferred_element_type=jnp.float32)
        au = jax.lax.bitcast_convert_type(
            zz[:16, :].astype(jnp.bfloat16), jnp.uint16).astype(jnp.uint32)
        bm = jax.lax.bitcast_convert_type(
            zz[16:, :].astype(jnp.bfloat16), jnp.uint16).astype(jnp.uint32)
        packed = (au | (bm << 16)).astype(jnp.int32)
        big = jnp.concatenate(
            [packed[:, i * 128:(i + 1) * 128] for i in range(8)],
            axis=0)
        zc_ref[pl.ds(g * 128, 128), :] = jnp.transpose(big)


@functools.cache
def _make_stage2():
    mesh = plsc.VectorSubcoreMesh(core_axis_name="c", subcore_axis_name="s",
                                  num_cores=NC, num_subcores=NS)

    @functools.partial(
        pl.kernel,
        out_type=jax.ShapeDtypeStruct((B,), jnp.float32),
        mesh=mesh,
        compiler_params=pltpu.CompilerParams(needs_layout_passes=False),
        scratch_types=[
            pltpu.VMEM((BPW,), jnp.int32),
            pltpu.VMEM((BPW,), jnp.int32),
            pltpu.VMEM((BPW,), jnp.int32),
            pltpu.VMEM((BPW,), jnp.int32),
            pltpu.VMEM((BPW,), jnp.int32),
            pltpu.VMEM((BPW,), jnp.int32),
            pltpu.VMEM((BPW // 4, 128), jnp.int32),
            pltpu.VMEM((BPW // 4, 128), jnp.int32),
            pltpu.VMEM((BPW, L), jnp.float32),
            pltpu.VMEM((BPW,), jnp.float32),
            pltpu.VMEM((L,), jnp.float32),
            pltpu.VMEM((L,), jnp.float32),
            pltpu.SemaphoreType.DMA,
            pltpu.SemaphoreType.DMA,
        ],
    )
    def _stage2(users_hbm, movies_hbm, zc_hbm, b1_hbm, w2_hbm,
                out_hbm, uidx_v, midx_v, urow_v, mrow_v, uslot_v, mslot_v,
                big_a, big_b, upiece_v, out_v, b1_v, w2_v, sem_a, sem_b):
        wid = lax.axis_index("s") * NC + lax.axis_index("c")
        base = wid * BPW
        pltpu.sync_copy(users_hbm.at[pl.ds(base, BPW)], uidx_v)
        pltpu.sync_copy(movies_hbm.at[pl.ds(base, BPW)], midx_v)
        pltpu.sync_copy(b1_hbm, b1_v)
        pltpu.sync_copy(w2_hbm, w2_v)

        def prep(ec, carry):
            ru = uidx_v[pl.ds(ec * L, L)]
            urow_v[pl.ds(ec * L, L)] = ((ru >> 10) << 7) + (ru & 127)
            uslot_v[pl.ds(ec * L, L)] = ((ru >> 7) & 7) << 4
            rm = midx_v[pl.ds(ec * L, L)]
            mrow_v[pl.ds(ec * L, L)] = ((rm >> 10) << 7) + (rm & 127)
            mslot_v[pl.ds(ec * L, L)] = ((rm >> 7) & 7) << 4
            return carry
        lax.fori_loop(0, BPW // L, prep, 0)

        HB = BPW // 4
        bufs = (big_a, big_b)
        sems = (sem_a, sem_b)

        cps = [pltpu.async_copy(zc_hbm.at[urow_v.at[pl.ds(h * HB, HB)]],
                                bufs[h], sems[h]) for h in range(2)]

        def make_extract_u(h0, big_v):
            def extract_u(ec, carry):
                svec = uslot_v[pl.ds(h0 + ec * L, L)]
                for k in range(L):
                    e = ec * L + k
                    chunk = plsc.bitcast(big_v[e, pl.ds(svec[k], L)],
                                         jnp.bfloat16)
                    zu_e, _ = plsc.unpack(chunk,
                                          format=plsc.PackFormat.INTERLEAVED)
                    upiece_v[h0 + e, :] = zu_e
                return carry
            return extract_u

        mcps = [None] * 4
        for q in range(4):
            b = q % 2
            cps[q].wait()
            lax.fori_loop(0, HB // L, make_extract_u(q * HB, bufs[b]), 0)
            if q < 2:
                cps.append(pltpu.async_copy(
                    zc_hbm.at[urow_v.at[pl.ds((q + 2) * HB, HB)]],
                    bufs[b], sems[b]))
            else:
                mcps[q - 2] = pltpu.async_copy(
                    zc_hbm.at[mrow_v.at[pl.ds((q - 2) * HB, HB)]],
                    bufs[b], sems[b])

        b1vec = b1_v[...]
        w2vec = w2_v[...]
        lanes = lax.iota(jnp.int32, L)

        def make_finish(h0, big_v):
            def finish(ec, carry):
                svec = mslot_v[pl.ds(h0 + ec * L, L)]
                acc = jnp.zeros((L,), jnp.float32)
                for k in range(L):
                    e = ec * L + k
                    chunk = plsc.bitcast(big_v[e, pl.ds(svec[k], L)],
                                         jnp.bfloat16)
                    _, mpiece = plsc.unpack(
                        chunk, format=plsc.PackFormat.INTERLEAVED)
                    h = jnp.maximum(upiece_v[h0 + e, :] + mpiece + b1vec, 0.0)
                    y = jnp.sum(h * w2vec, axis=0)
                    acc = jnp.where(lanes == k, y, acc)
                out_v[pl.ds(h0 + ec * L, L)] = \
                    6.0 / (1.0 + jnp.exp(-acc)) - 0.5
                return carry
            return finish

        for q in range(4):
            b = q % 2
            mcps[q].wait()
            lax.fori_loop(0, HB // L, make_finish(q * HB, bufs[b]), 0)
            if q < 2:
                mcps[q + 2] = pltpu.async_copy(
                    zc_hbm.at[mrow_v.at[pl.ds((q + 2) * HB, HB)]],
                    bufs[b], sems[b])

        pltpu.sync_copy(out_v, out_hbm.at[pl.ds(base, BPW)])

    return _stage2


def kernel(users, movies, U, M, W1, b1, W2, b2):
    users = users.astype(jnp.int32)
    movies = movies.astype(jnp.int32)

    w1c = (jnp.zeros((2 * L, 2 * D), jnp.float32)
           .at[:10, :D].set(W1[:, :D])
           .at[L:L + 10, D:].set(W1[:, D:]))
    b1p = jnp.zeros((L,), jnp.float32).at[:10].set(b1).at[10].set(1.0)
    w2p = jnp.zeros((L,), jnp.float32).at[:10].set(W2[0]).at[10].set(b2[0])

    zc = pl.pallas_call(
        _stage1_body,
        grid=(NBLK,),
        in_specs=[
            pl.BlockSpec((D, BLK), lambda i: (0, i)),
            pl.BlockSpec((D, BLK), lambda i: (0, i)),
            pl.BlockSpec((2 * L, 2 * D), lambda i: (0, 0)),
        ],
        out_specs=pl.BlockSpec((BLK // 8, 128), lambda i: (i, 0)),
        out_shape=jax.ShapeDtypeStruct((NZ, 128), jnp.int32),
    )(U.T, M.T, w1c)

    return _make_stage2()(users, movies, zc, b1p, w2p)


# speedup vs baseline: 5.5364x; 1.1040x over previous
import functools

import jax
import jax.numpy as jnp
from jax import lax
from jax.experimental import pallas as pl
from jax.experimental.pallas import tpu as pltpu
from jax.experimental.pallas import tpu_sc as plsc

B = 16384
D = 64
R = 1_000_000
NC, NS = 2, 16
NW = NC * NS
BPW = B // NW
L = 16
BLK = 16384
NBLK = (R + BLK - 1) // BLK
NZ = NBLK * (BLK // 8)


def _stage1_body(ut_ref, mt_ref, w1_ref, zc_ref):
    for g in range(BLK // 1024):
        cols = pl.ds(g * 1024, 1024)
        zz = jnp.dot(w1_ref[...],
                     jnp.concatenate([ut_ref[:, cols], mt_ref[:, cols]],
                                     axis=0),
                     preferred_element_type=jnp.float32)
        au = jax.lax.bitcast_convert_type(
            zz[:16, :].astype(jnp.bfloat16), jnp.uint16).astype(jnp.uint32)
        bm = jax.lax.bitcast_convert_type(
            zz[16:, :].astype(jnp.bfloat16), jnp.uint16).astype(jnp.uint32)
        packed = (au | (bm << 16)).astype(jnp.int32)
        big = jnp.concatenate(
            [packed[:, i * 128:(i + 1) * 128] for i in range(8)],
            axis=0)
        zc_ref[pl.ds(g * 128, 128), :] = jnp.transpose(big)


@functools.cache
def _make_stage2():
    mesh = plsc.VectorSubcoreMesh(core_axis_name="c", subcore_axis_name="s",
                                  num_cores=NC, num_subcores=NS)

    @functools.partial(
        pl.kernel,
        out_type=jax.ShapeDtypeStruct((B,), jnp.float32),
        mesh=mesh,
        compiler_params=pltpu.CompilerParams(needs_layout_passes=False),
        scratch_types=[
            pltpu.VMEM((BPW,), jnp.int32),
            pltpu.VMEM((BPW,), jnp.int32),
            pltpu.VMEM((BPW,), jnp.int32),
            pltpu.VMEM((BPW,), jnp.int32),
            pltpu.VMEM((BPW,), jnp.int32),
            pltpu.VMEM((BPW,), jnp.int32),
            pltpu.VMEM((BPW // 4, 128), jnp.int32),
            pltpu.VMEM((BPW // 4, 128), jnp.int32),
            pltpu.VMEM((BPW, L), jnp.float32),
            pltpu.VMEM((BPW,), jnp.float32),
            pltpu.VMEM((L,), jnp.float32),
            pltpu.VMEM((L,), jnp.float32),
            pltpu.SemaphoreType.DMA,
            pltpu.SemaphoreType.DMA,
        ],
    )
    def _stage2(users_hbm, movies_hbm, zc_hbm, b1_hbm, w2_hbm,
                out_hbm, uidx_v, midx_v, urow_v, mrow_v, uslot_v, mslot_v,
                big_a, big_b, upiece_v, out_v, b1_v, w2_v, sem_a, sem_b):
        wid = lax.axis_index("s") * NC + lax.axis_index("c")
        base = wid * BPW
        pltpu.sync_copy(users_hbm.at[pl.ds(base, BPW)], uidx_v)
        pltpu.sync_copy(movies_hbm.at[pl.ds(base, BPW)], midx_v)
        pltpu.sync_copy(b1_hbm, b1_v)
        pltpu.sync_copy(w2_hbm, w2_v)

        def prep(ec, carry):
            ru = uidx_v[pl.ds(ec * L, L)]
            urow_v[pl.ds(ec * L, L)] = ((ru >> 10) << 7) + (ru & 127)
            uslot_v[pl.ds(ec * L, L)] = ((ru >> 7) & 7) << 4
            rm = midx_v[pl.ds(ec * L, L)]
            mrow_v[pl.ds(ec * L, L)] = ((rm >> 10) << 7) + (rm & 127)
            mslot_v[pl.ds(ec * L, L)] = ((rm >> 7) & 7) << 4
            return carry
        lax.fori_loop(0, BPW // L, prep, 0)

        HB = BPW // 4
        bufs = (big_a, big_b)
        sems = (sem_a, sem_b)

        cps = [pltpu.async_copy(zc_hbm.at[urow_v.at[pl.ds(h * HB, HB)]],
                                bufs[h], sems[h]) for h in range(2)]

        def make_extract_u(h0, big_v):
            def extract_u(ec, carry):
                svec = uslot_v[pl.ds(h0 + ec * L, L)]
                for k in range(L):
                    e = ec * L + k
                    chunk = plsc.bitcast(big_v[e, pl.ds(svec[k], L)],
                                         jnp.bfloat16)
                    zu_e, _ = plsc.unpack(chunk,
                                          format=plsc.PackFormat.INTERLEAVED)
                    upiece_v[h0 + e, :] = zu_e
                return carry
            return extract_u

        mcps = [None] * 4
        for q in range(4):
            b = q % 2
            cps[q].wait()
            lax.fori_loop(0, HB // L, make_extract_u(q * HB, bufs[b]), 0)
            if q < 2:
                cps.append(pltpu.async_copy(
                    zc_hbm.at[urow_v.at[pl.ds((q + 2) * HB, HB)]],
                    bufs[b], sems[b]))
            else:
                mcps[q - 2] = pltpu.async_copy(
                    zc_hbm.at[mrow_v.at[pl.ds((q - 2) * HB, HB)]],
                    bufs[b], sems[b])

        b1vec = b1_v[...]
        w2vec = w2_v[...]
        lanes = lax.iota(jnp.int32, L)

        def make_finish(h0, big_v):
            def finish(ec, carry):
                svec = mslot_v[pl.ds(h0 + ec * L, L)]
                acc = jnp.zeros((L,), jnp.float32)
                for k in range(L):
                    e = ec * L + k
                    chunk = plsc.bitcast(big_v[e, pl.ds(svec[k], L)],
                                         jnp.bfloat16)
                    _, mpiece = plsc.unpack(
                        chunk, format=plsc.PackFormat.INTERLEAVED)
                    h = jnp.maximum(upiece_v[h0 + e, :] + mpiece + b1vec, 0.0)
                    y = jnp.sum(h * w2vec, axis=0)
                    acc = jnp.where(lanes == k, y, acc)
                out_v[pl.ds(h0 + ec * L, L)] = \
                    6.0 / (1.0 + jnp.exp(-acc)) - 0.5
                return carry
            return finish

        for q in range(4):
            b = q % 2
            mcps[q].wait()
            lax.fori_loop(0, HB // L, make_finish(q * HB, bufs[b]), 0)
            if q < 2:
                mcps[q + 2] = pltpu.async_copy(
                    zc_hbm.at[mrow_v.at[pl.ds((q + 2) * HB, HB)]],
                    bufs[b], sems[b])

        pltpu.sync_copy(out_v, out_hbm.at[pl.ds(base, BPW)])

    return _stage2


def kernel(users, movies, U, M, W1, b1, W2, b2):
    users = users.astype(jnp.int32)
    movies = movies.astype(jnp.int32)

    w1c = (jnp.zeros((2 * L, 2 * D), jnp.float32)
           .at[:10, :D].set(W1[:, :D])
           .at[L:L + 10, D:].set(W1[:, D:]))
    b1p = jnp.zeros((L,), jnp.float32).at[:10].set(b1).at[10].set(1.0)
    w2p = jnp.zeros((L,), jnp.float32).at[:10].set(W2[0]).at[10].set(b2[0])

    zc = pl.pallas_call(
        _stage1_body,
        grid=(NBLK,),
        in_specs=[
            pl.BlockSpec((D, BLK), lambda i: (0, i)),
            pl.BlockSpec((D, BLK), lambda i: (0, i)),
            pl.BlockSpec((2 * L, 2 * D), lambda i: (0, 0)),
        ],
        out_specs=pl.BlockSpec((BLK // 8, 128), lambda i: (i, 0)),
        out_shape=jax.ShapeDtypeStruct((NZ, 128), jnp.int32),
    )(U.T, M.T, w1c)

    return _make_stage2()(users, movies, zc, b1p, w2p)


# speedup vs baseline: 5.5735x; 1.0067x over previous
import functools

import jax
import jax.numpy as jnp
from jax import lax
from jax.experimental import pallas as pl
from jax.experimental.pallas import tpu as pltpu
from jax.experimental.pallas import tpu_sc as plsc

B = 16384
D = 64
R = 1_000_000
NC, NS = 2, 16
NW = NC * NS
BPW = B // NW
L = 16
BLK = 32768
NBLK = (R + BLK - 1) // BLK
NZ = NBLK * (BLK // 8)


def _stage1_body(ut_ref, mt_ref, w1_ref, zc_ref):
    for g in range(BLK // 1024):
        cols = pl.ds(g * 1024, 1024)
        zz = jnp.dot(w1_ref[...],
                     jnp.concatenate([ut_ref[:, cols], mt_ref[:, cols]],
                                     axis=0),
                     preferred_element_type=jnp.float32)
        au = jax.lax.bitcast_convert_type(
            zz[:16, :].astype(jnp.bfloat16), jnp.uint16).astype(jnp.uint32)
        bm = jax.lax.bitcast_convert_type(
            zz[16:, :].astype(jnp.bfloat16), jnp.uint16).astype(jnp.uint32)
        packed = (au | (bm << 16)).astype(jnp.int32)
        big = jnp.concatenate(
            [packed[:, i * 128:(i + 1) * 128] for i in range(8)],
            axis=0)
        zc_ref[pl.ds(g * 128, 128), :] = jnp.transpose(big)


@functools.cache
def _make_stage2():
    mesh = plsc.VectorSubcoreMesh(core_axis_name="c", subcore_axis_name="s",
                                  num_cores=NC, num_subcores=NS)

    @functools.partial(
        pl.kernel,
        out_type=jax.ShapeDtypeStruct((B,), jnp.float32),
        mesh=mesh,
        compiler_params=pltpu.CompilerParams(needs_layout_passes=False),
        scratch_types=[
            pltpu.VMEM((BPW,), jnp.int32),
            pltpu.VMEM((BPW,), jnp.int32),
            pltpu.VMEM((BPW,), jnp.int32),
            pltpu.VMEM((BPW,), jnp.int32),
            pltpu.VMEM((BPW,), jnp.int32),
            pltpu.VMEM((BPW,), jnp.int32),
            pltpu.VMEM((BPW // 4, 128), jnp.int32),
            pltpu.VMEM((BPW // 4, 128), jnp.int32),
            pltpu.VMEM((BPW, L), jnp.float32),
            pltpu.VMEM((BPW,), jnp.float32),
            pltpu.VMEM((L,), jnp.float32),
            pltpu.VMEM((L,), jnp.float32),
            pltpu.SemaphoreType.DMA,
            pltpu.SemaphoreType.DMA,
        ],
    )
    def _stage2(users_hbm, movies_hbm, zc_hbm, b1_hbm, w2_hbm,
                out_hbm, uidx_v, midx_v, urow_v, mrow_v, uslot_v, mslot_v,
                big_a, big_b, upiece_v, out_v, b1_v, w2_v, sem_a, sem_b):
        wid = lax.axis_index("s") * NC + lax.axis_index("c")
        base = wid * BPW
        pltpu.sync_copy(users_hbm.at[pl.ds(base, BPW)], uidx_v)
        pltpu.sync_copy(movies_hbm.at[pl.ds(base, BPW)], midx_v)
        pltpu.sync_copy(b1_hbm, b1_v)
        pltpu.sync_copy(w2_hbm, w2_v)

        def prep(ec, carry):
            ru = uidx_v[pl.ds(ec * L, L)]
            urow_v[pl.ds(ec * L, L)] = ((ru >> 10) << 7) + (ru & 127)
            uslot_v[pl.ds(ec * L, L)] = ((ru >> 7) & 7) << 4
            rm = midx_v[pl.ds(ec * L, L)]
            mrow_v[pl.ds(ec * L, L)] = ((rm >> 10) << 7) + (rm & 127)
            mslot_v[pl.ds(ec * L, L)] = ((rm >> 7) & 7) << 4
            return carry
        lax.fori_loop(0, BPW // L, prep, 0)

        HB = BPW // 4
        bufs = (big_a, big_b)
        sems = (sem_a, sem_b)

        cps = [pltpu.async_copy(zc_hbm.at[urow_v.at[pl.ds(h * HB, HB)]],
                                bufs[h], sems[h]) for h in range(2)]

        def make_extract_u(h0, big_v):
            def extract_u(ec, carry):
                svec = uslot_v[pl.ds(h0 + ec * L, L)]
                for k in range(L):
                    e = ec * L + k
                    chunk = plsc.bitcast(big_v[e, pl.ds(svec[k], L)],
                                         jnp.bfloat16)
                    zu_e, _ = plsc.unpack(chunk,
                                          format=plsc.PackFormat.INTERLEAVED)
                    upiece_v[h0 + e, :] = zu_e
                return carry
            return extract_u

        mcps = [None] * 4
        for q in range(4):
            b = q % 2
            cps[q].wait()
            lax.fori_loop(0, HB // L, make_extract_u(q * HB, bufs[b]), 0)
            if q < 2:
                cps.append(pltpu.async_copy(
                    zc_hbm.at[urow_v.at[pl.ds((q + 2) * HB, HB)]],
                    bufs[b], sems[b]))
            else:
                mcps[q - 2] = pltpu.async_copy(
                    zc_hbm.at[mrow_v.at[pl.ds((q - 2) * HB, HB)]],
                    bufs[b], sems[b])

        b1vec = b1_v[...]
        w2vec = w2_v[...]
        lanes = lax.iota(jnp.int32, L)

        def make_finish(h0, big_v):
            def finish(ec, carry):
                svec = mslot_v[pl.ds(h0 + ec * L, L)]
                acc = jnp.zeros((L,), jnp.float32)
                for k in range(L):
                    e = ec * L + k
                    chunk = plsc.bitcast(big_v[e, pl.ds(svec[k], L)],
                                         jnp.bfloat16)
                    _, mpiece = plsc.unpack(
                        chunk, format=plsc.PackFormat.INTERLEAVED)
                    h = jnp.maximum(upiece_v[h0 + e, :] + mpiece + b1vec, 0.0)
                    y = jnp.sum(h * w2vec, axis=0)
                    acc = jnp.where(lanes == k, y, acc)
                out_v[pl.ds(h0 + ec * L, L)] = \
                    6.0 / (1.0 + jnp.exp(-acc)) - 0.5
                return carry
            return finish

        for q in range(4):
            b = q % 2
            mcps[q].wait()
            lax.fori_loop(0, HB // L, make_finish(q * HB, bufs[b]), 0)
            if q < 2:
                mcps[q + 2] = pltpu.async_copy(
                    zc_hbm.at[mrow_v.at[pl.ds((q + 2) * HB, HB)]],
                    bufs[b], sems[b])

        pltpu.sync_copy(out_v, out_hbm.at[pl.ds(base, BPW)])

    return _stage2


def kernel(users, movies, U, M, W1, b1, W2, b2):
    users = users.astype(jnp.int32)
    movies = movies.astype(jnp.int32)

    w1c = (jnp.zeros((2 * L, 2 * D), jnp.float32)
           .at[:10, :D].set(W1[:, :D])
           .at[L:L + 10, D:].set(W1[:, D:]))
    b1p = jnp.zeros((L,), jnp.float32).at[:10].set(b1).at[10].set(1.0)
    w2p = jnp.zeros((L,), jnp.float32).at[:10].set(W2[0]).at[10].set(b2[0])

    zc = pl.pallas_call(
        _stage1_body,
        grid=(NBLK,),
        in_specs=[
            pl.BlockSpec((D, BLK), lambda i: (0, i)),
            pl.BlockSpec((D, BLK), lambda i: (0, i)),
            pl.BlockSpec((2 * L, 2 * D), lambda i: (0, 0)),
        ],
        out_specs=pl.BlockSpec((BLK // 8, 128), lambda i: (i, 0)),
        out_shape=jax.ShapeDtypeStruct((NZ, 128), jnp.int32),
    )(U.T, M.T, w1c)

    return _make_stage2()(users, movies, zc, b1p, w2p)
